# baseline (speedup 1.0000x reference)
import jax
import jax.numpy as jnp
from jax.experimental import pallas as pl
from jax.experimental.pallas import tpu as pltpu


def _make_fused_body(nb, slab, nchunks, ch):
    def _body(w_ref, x_ref, mem_ref, q_ref, out_ref, ret_ref, conf_ref):
        i = pl.program_id(0)

        @pl.when(i == 0)
        def _():
            out_ref[...] = jnp.zeros_like(out_ref)

        @pl.when(i < nb)
        def _():
            s = jnp.sum(w_ref[...], axis=1, keepdims=True)
            xs = x_ref[...] / s
            out_ref[...] += jax.lax.dot_general(
                w_ref[...], xs,
                dimension_numbers=(((0,), (0,)), ((), ())),
                preferred_element_type=jnp.float32)
            out_ref[pl.ds(jnp.minimum(i, nb - 1) * slab, slab), :] += (
                mem_ref[...])

        @pl.when(i >= nb)
        def _():
            q = q_ref[...]
            ms, ls, accs = [], [], []
            for c in range(nchunks):
                memc = out_ref[c * ch:(c + 1) * ch, :]
                sc = jax.lax.dot_general(
                    q, memc,
                    dimension_numbers=(((1,), (1,)), ((), ())),
                    preferred_element_type=jnp.float32)
                mc = jnp.max(sc, axis=1, keepdims=True)
                p = jnp.exp(sc - mc)
                ms.append(mc)
                ls.append(jnp.sum(p, axis=1, keepdims=True))
                accs.append(jax.lax.dot_general(
                    p, memc,
                    dimension_numbers=(((1,), (0,)), ((), ())),
                    preferred_element_type=jnp.float32))

            m_new = ms[0]
            for mc in ms[1:]:
                m_new = jnp.maximum(m_new, mc)
            l = jnp.zeros_like(ls[0])
            acc = jnp.zeros_like(accs[0])
            for mc, lc, ac in zip(ms, ls, accs):
                wc = jnp.exp(mc - m_new)
                l += lc * wc
                acc += ac * wc
            linv = 1.0 / l
            ret_ref[...] = acc * linv
            conf_ref[...] = linv
    return _body


def kernel(memory, input_data, write_weights, query):
    M, F = memory.shape
    B = input_data.shape[0]
    f32 = jnp.float32

    BB = min(64, B)
    nb = B // BB
    slab = M // nb
    BQ = min(256, B)
    nq = B // BQ
    CH = min(2048, M)
    nchunks = M // CH

    wlast = nb - 1
    memory_new, retrieved, conf = pl.pallas_call(
        _make_fused_body(nb, slab, nchunks, CH),
        grid=(nb + nq,),
        in_specs=[
            pl.BlockSpec((BB, M), lambda i: (jnp.minimum(i, wlast), 0)),
            pl.BlockSpec((BB, F), lambda i: (jnp.minimum(i, wlast), 0)),
            pl.BlockSpec((slab, F), lambda i: (jnp.minimum(i, wlast), 0)),
            pl.BlockSpec((BQ, F), lambda i: (jnp.maximum(i - nb, 0), 0)),
        ],
        out_specs=[
            pl.BlockSpec((M, F), lambda i: (0, 0)),
            pl.BlockSpec((BQ, F), lambda i: (jnp.maximum(i - nb, 0), 0)),
            pl.BlockSpec((BQ, 1), lambda i: (jnp.maximum(i - nb, 0), 0)),
        ],
        out_shape=[
            jax.ShapeDtypeStruct((M, F), f32),
            jax.ShapeDtypeStruct((B, F), f32),
            jax.ShapeDtypeStruct((B, 1), f32),
        ],
        compiler_params=pltpu.CompilerParams(
            dimension_semantics=("arbitrary",),
            vmem_limit_bytes=56 * 1024 * 1024),
        name="memory_bank_fused",
    )(write_weights, input_data, memory, query)

    return retrieved, conf[:, 0], memory_new


# speedup vs baseline: 1.3629x; 1.3629x over previous
import jax
import jax.numpy as jnp
from jax.experimental import pallas as pl
from jax.experimental.pallas import tpu as pltpu


def _make_write_body(slab):
    def _write_body(w_ref, x_ref, mem_ref, out_ref):
        i = pl.program_id(0)

        @pl.when(i == 0)
        def _():
            out_ref[...] = jnp.zeros_like(out_ref)

        s = jnp.sum(w_ref[...], axis=1, keepdims=True)
        xs = x_ref[...] / s
        out_ref[...] += jax.lax.dot_general(
            w_ref[...], xs,
            dimension_numbers=(((0,), (0,)), ((), ())),
            preferred_element_type=jnp.float32)
        out_ref[pl.ds(i * slab, slab), :] += mem_ref[...]
    return _write_body


def _make_attn_body(nchunks, ch):
    def _attn_body(q_ref, mem_ref, out_ref, conf_ref, acc_ref, m_ref, l_ref):
        j = pl.program_id(1)

        @pl.when(j == 0)
        def _():
            acc_ref[...] = jnp.zeros_like(acc_ref)
            m_ref[...] = jnp.full_like(m_ref, -jnp.inf)
            l_ref[...] = jnp.zeros_like(l_ref)

        q = q_ref[...]
        ms, ls, accs = [], [], []
        for c in range(nchunks):
            memc = mem_ref[c * ch:(c + 1) * ch, :]
            s = jax.lax.dot_general(
                q, memc,
                dimension_numbers=(((1,), (1,)), ((), ())),
                preferred_element_type=jnp.float32)
            mc = jnp.max(s, axis=1, keepdims=True)
            p = jnp.exp(s - mc)
            ms.append(mc)
            ls.append(jnp.sum(p, axis=1, keepdims=True))
            accs.append(jax.lax.dot_general(
                p, memc,
                dimension_numbers=(((1,), (0,)), ((), ())),
                preferred_element_type=jnp.float32))

        m_prev = m_ref[...]
        m_new = m_prev
        for mc in ms:
            m_new = jnp.maximum(m_new, mc)
        alpha = jnp.exp(m_prev - m_new)
        l = l_ref[...] * alpha
        acc = acc_ref[...] * alpha
        for mc, lc, ac in zip(ms, ls, accs):
            wc = jnp.exp(mc - m_new)
            l += lc * wc
            acc += ac * wc
        m_ref[...] = m_new
        l_ref[...] = l
        acc_ref[...] = acc

        @pl.when(j == pl.num_programs(1) - 1)
        def _():
            linv = 1.0 / l_ref[...]
            out_ref[...] = acc_ref[...] * linv
            conf_ref[...] = linv
    return _attn_body


def kernel(memory, input_data, write_weights, query):
    M, F = memory.shape
    B = input_data.shape[0]
    f32 = jnp.float32

    BB = min(128, B)
    nb = B // BB
    slab = M // nb
    memory_new = pl.pallas_call(
        _make_write_body(slab),
        grid=(nb,),
        in_specs=[pl.BlockSpec((BB, M), lambda i: (i, 0)),
                  pl.BlockSpec((BB, F), lambda i: (i, 0)),
                  pl.BlockSpec((slab, F), lambda i: (i, 0))],
        out_specs=pl.BlockSpec((M, F), lambda i: (0, 0)),
        out_shape=jax.ShapeDtypeStruct((M, F), f32),
        compiler_params=pltpu.CompilerParams(
            dimension_semantics=("arbitrary",),
            vmem_limit_bytes=56 * 1024 * 1024),
        name="memory_update",
    )(write_weights, input_data, memory)

    BQ, BK = min(512, B), min(32768, M)
    CH = min(2048, BK)
    retrieved, conf = pl.pallas_call(
        _make_attn_body(BK // CH, CH),
        grid=(B // BQ, M // BK),
        in_specs=[pl.BlockSpec((BQ, F), lambda i, j: (i, 0)),
                  pl.BlockSpec((BK, F), lambda i, j: (j, 0))],
        out_specs=[pl.BlockSpec((BQ, F), lambda i, j: (i, 0)),
                   pl.BlockSpec((BQ, 1), lambda i, j: (i, 0))],
        out_shape=[jax.ShapeDtypeStruct((B, F), f32),
                   jax.ShapeDtypeStruct((B, 1), f32)],
        scratch_shapes=[pltpu.VMEM((BQ, F), f32),
                        pltpu.VMEM((BQ, 1), f32),
                        pltpu.VMEM((BQ, 1), f32)],
        compiler_params=pltpu.CompilerParams(
            dimension_semantics=("parallel", "arbitrary"),
            vmem_limit_bytes=56 * 1024 * 1024),
        name="attention_read",
    )(query, memory_new)

    return retrieved, conf[:, 0], memory_new
